# Initial kernel scaffold; baseline (speedup 1.0000x reference)
import jax
import jax.numpy as jnp
from jax.experimental import pallas as pl


def kernel(x, weights):
    raise NotImplementedError("write your pallas kernel here")


# speedup vs baseline: 2.5420x; 2.5420x over previous
import functools

import numpy as np
import jax
import jax.numpy as jnp
from jax import lax
from jax.experimental import pallas as pl
from jax.experimental.pallas import tpu as pltpu
from jax.experimental.pallas import tpu_sc as plsc

_N = 64
_MEM = 6
_B = 1024
_L = 16
_NW = 32
_BPW = _B // _NW
_NCH = _BPW // _L
_STEPS = 128
_OUT_STEPS = 65
_CLAMP = 50.0
_INIT = 20.0


def _branch_sign_structure():
    gm = np.array([[1, 1, 1, 1, 0, 0, 1], [1, 0, 1, 1, 0, 1, 1]], dtype=np.int64)
    s = np.arange(_N)[:, None]
    b = np.arange(2)[None, :]
    p = 2 * (s % (_N // 2)) + b
    u = np.broadcast_to(s >> (_MEM - 1), p.shape)
    bits = np.zeros((_N, 2, _MEM + 1), dtype=np.int64)
    bits[:, :, 0] = u
    for j in range(_MEM):
        bits[:, :, j + 1] = (p >> (_MEM - 1 - j)) & 1
    c = np.einsum('rk,sbk->rsb', gm, bits) % 2
    signs = 1.0 - 2.0 * c
    s00, s10 = signs[0, :, 0], signs[1, :, 0]
    assert np.all(signs[0, :, 1] == -s00) and np.all(signs[1, :, 1] == -s10)
    use_sum = [bool(s00[i] == s10[i]) for i in range(_N)]
    positive = [bool(s00[i] > 0) for i in range(_N)]
    return use_sum, positive


_USE_SUM, _POSITIVE = _branch_sign_structure()


def _acs_step(x_vm, w_vm, src, dst, col, wrow):
    wvec = [w_vm[pl.ds(wrow * _N + g * _L, _L)] for g in range(_N // _L)]
    ws = [wvec[s >> 4][s & 15] for s in range(_N)]
    for ch in range(_NCH):
        lo = ch * _L
        x0 = x_vm[pl.ds(col * _BPW + lo, _L)]
        x1 = x_vm[pl.ds(col * _BPW + _BPW + lo, _L)]
        asum = x0 + x1
        adif = x0 - x1
        sums = [None, None, None, None]
        for m in range(_N // 2):
            p0 = src[pl.ds(2 * m * _BPW + lo, _L)]
            p1 = src[pl.ds((2 * m + 1) * _BPW + lo, _L)]
            for s in (m, m + _N // 2):
                t = ws[s] * (asum if _USE_SUM[s] else adif)
                if _POSITIVE[s]:
                    o = jnp.maximum(p0 + t, p1 - t)
                else:
                    o = jnp.maximum(p0 - t, p1 + t)
                dst[pl.ds(s * _BPW + lo, _L)] = o
                j = s & 3
                sums[j] = o if sums[j] is None else sums[j] + o
        mean = ((sums[0] + sums[1]) + (sums[2] + sums[3])) * (1.0 / _N)
        for s in range(_N):
            v = dst[pl.ds(s * _BPW + lo, _L)] - mean
            dst[pl.ds(s * _BPW + lo, _L)] = jnp.minimum(
                jnp.maximum(v, -_CLAMP), _CLAMP)


_XW = _STEPS * _BPW
_OW = _N * _BPW


def _sc_decode(x_in, w_in):
    mesh = plsc.VectorSubcoreMesh(core_axis_name="c", subcore_axis_name="s")

    @functools.partial(
        pl.kernel,
        mesh=mesh,
        out_type=jax.ShapeDtypeStruct((_NW * _OUT_STEPS * _OW,), jnp.float32),
        scratch_types=[
            pltpu.VMEM((_STEPS * _BPW,), jnp.float32),
            pltpu.VMEM((_STEPS * _N,), jnp.float32),
            pltpu.VMEM((_N * _BPW,), jnp.float32),
            pltpu.VMEM((_N * _BPW,), jnp.float32),
            pltpu.SemaphoreType.DMA,
            pltpu.SemaphoreType.DMA,
        ],
    )
    def k(x_hbm, w_hbm, out_hbm, x_vm, w_vm, pa, pb, sem_a, sem_b):
        wid = lax.axis_index("s") * 2 + lax.axis_index("c")
        obase = wid * (_OUT_STEPS * _OW)

        def orow(row):
            return out_hbm.at[pl.ds(pl.multiple_of(obase + row * _OW, _OW), _OW)]

        pltpu.sync_copy(
            x_hbm.at[pl.ds(pl.multiple_of(wid * _XW, _XW), _XW)], x_vm)
        pltpu.sync_copy(w_hbm, w_vm)
        init = jnp.full((_L,), _INIT, jnp.float32)
        zero = jnp.zeros((_L,), jnp.float32)
        for ch in range(_NCH):
            pa[pl.ds(ch * _L, _L)] = init
            for s in range(1, _N):
                pa[pl.ds(s * _BPW + ch * _L, _L)] = zero

        def body(kk, carry):
            ca = (4 * kk) & 127
            @pl.when(kk >= 33)
            def _():
                pltpu.make_async_copy(pb, orow(2 * kk - 65), sem_b).wait()
            _acs_step(x_vm, w_vm, pa, pb, ca, 2 * kk)
            @pl.when(kk >= 32)
            def _():
                pltpu.async_copy(pb, orow(2 * kk - 63), sem_b)
            cb = (4 * kk + 2) & 127
            @pl.when(kk >= 32)
            def _():
                pltpu.make_async_copy(pa, orow(2 * kk - 64), sem_a).wait()
            _acs_step(x_vm, w_vm, pb, pa, cb, 2 * kk + 1)
            @pl.when(kk >= 31)
            def _():
                pltpu.async_copy(pa, orow(2 * kk - 62), sem_a)
            return carry

        lax.fori_loop(0, _STEPS // 2, body, 0)
        pltpu.make_async_copy(pb, orow(63), sem_b).wait()
        pltpu.make_async_copy(pa, orow(64), sem_a).wait()

    return k(x_in, w_in)


def kernel(x, weights):
    x_in = (x.T.reshape(_STEPS, _NW, _BPW)
            .transpose(1, 0, 2).reshape(_NW * _STEPS * _BPW))
    w_eff = jnp.concatenate(
        [jnp.ones((_STEPS - _OUT_STEPS, _N), jnp.float32),
         weights[_STEPS - _OUT_STEPS:_STEPS]], axis=0)
    out = _sc_decode(x_in, w_eff.reshape(-1))
    return (out.reshape(_NW, _OUT_STEPS, _N, _BPW)
            .transpose(0, 3, 1, 2).reshape(_B, _OUT_STEPS * _N))
